# baseline (speedup 1.0000x reference)
import functools

import jax
import jax.numpy as jnp
from jax import lax
from jax.experimental import pallas as pl
from jax.experimental.pallas import tpu as pltpu
from jax.experimental.pallas import tpu_sc as plsc

NUM_USER = 100000
NUM_ITEMS = 100000
LD = 16
CD = 32
FD = 64
B, J, K = 1024, 20, 50
JK = J * K
N_TOT = B * JK
HB = JK // 2
N_PK = N_TOT // 2

NC, NS = 2, 16
NW = NC * NS
PER_W = N_TOT // NW
SUB = (128, 128, 64)
GROUP = sum(SUB)
N_GROUPS = PER_W // GROUP
U_PER_W = B // NW

BB = 8


def _sc_gather(table, flat_idx):
    mesh = plsc.VectorSubcoreMesh(core_axis_name="c", subcore_axis_name="s")

    @functools.partial(
        pl.kernel,
        out_type=jax.ShapeDtypeStruct((N_TOT, FD), jnp.float32),
        mesh=mesh,
        scratch_types=[
            pltpu.VMEM((PER_W,), jnp.int32),
            pltpu.VMEM((2, GROUP, FD), jnp.float32),
            pltpu.SemaphoreType.DMA,
            pltpu.SemaphoreType.DMA,
            pltpu.SemaphoreType.DMA,
            pltpu.SemaphoreType.DMA,
        ],
        compiler_params=pltpu.CompilerParams(use_tc_tiling_on_sc=False),
    )
    def gather_kernel(table_hbm, idx_hbm, out_hbm,
                      idx_v, rows_v, gsem0, gsem1, wsem0, wsem1):
        wid = lax.axis_index("s") * NC + lax.axis_index("c")
        gsems = (gsem0, gsem1)
        wsems = (wsem0, wsem1)

        wbase = pl.multiple_of(wid * PER_W, 8)
        pltpu.sync_copy(idx_hbm.at[pl.ds(wbase, PER_W)], idx_v)

        def issue_gather(g, s):
            off = 0
            for n in SUB:
                goff = pl.multiple_of(g * GROUP + off, 8)
                sl = idx_v.at[pl.ds(goff, n)]
                pltpu.async_copy(table_hbm.at[sl],
                                 rows_v.at[s, pl.ds(off, n)],
                                 gsems[s])
                off += n

        def drain_gather(s):
            pltpu.make_async_copy(table_hbm.at[pl.ds(0, GROUP)], rows_v.at[s],
                                  gsems[s]).wait()

        def issue_writeback(g, s):
            base = pl.multiple_of(wbase + g * GROUP, 8)
            pltpu.async_copy(rows_v.at[s], out_hbm.at[pl.ds(base, GROUP)],
                             wsems[s])

        def drain_writeback(s):
            pltpu.make_async_copy(table_hbm.at[pl.ds(0, GROUP)], rows_v.at[s],
                                  wsems[s]).wait()

        issue_gather(0, 0)

        def body(t, carry):
            for s in range(2):
                g = t * 2 + s
                drain_gather(s)
                issue_writeback(g, s)

                @pl.when(jnp.logical_and(g >= 1, g < N_GROUPS - 1))
                def _():
                    drain_writeback(1 - s)

                @pl.when(g < N_GROUPS - 1)
                def _():
                    issue_gather(g + 1, 1 - s)

            return carry

        lax.fori_loop(0, N_GROUPS // 2, body, 0)
        drain_writeback(0)
        drain_writeback(1)

    return gather_kernel(table, flat_idx)


def _sc_gather_u(u_flat, user_idx):
    mesh = plsc.VectorSubcoreMesh(core_axis_name="c", subcore_axis_name="s")

    @functools.partial(
        pl.kernel,
        out_type=jax.ShapeDtypeStruct((B, LD * CD), jnp.float32),
        mesh=mesh,
        scratch_types=[
            pltpu.VMEM((U_PER_W,), jnp.int32),
            pltpu.VMEM((U_PER_W, LD * CD), jnp.float32),
            pltpu.SemaphoreType.DMA,
        ],
    )
    def u_kernel(u_hbm, uidx_hbm, usel_hbm, uidx_v, urows_v, sem):
        wid = lax.axis_index("s") * NC + lax.axis_index("c")
        ubase = pl.multiple_of(wid * U_PER_W, 8)
        pltpu.sync_copy(uidx_hbm.at[pl.ds(ubase, U_PER_W)], uidx_v)
        pltpu.async_copy(u_hbm.at[uidx_v], urows_v, sem).wait()
        pltpu.sync_copy(urows_v, usel_hbm.at[pl.ds(ubase, U_PER_W)])

    return u_kernel(u_flat, user_idx)


def _tc_compute(gpk, mp, u_selT):
    def body(g_ref, mp_ref, u_ref, out_ref):
        rows_j = lax.broadcasted_iota(jnp.int32, (HB, J), 0) // K
        cols_j = lax.broadcasted_iota(jnp.int32, (HB, J), 1)
        expand = jnp.logical_or(rows_j == cols_j,
                                rows_j + J // 2 == cols_j).astype(jnp.float32)
        lane = lax.broadcasted_iota(jnp.int32, (128, 2), 0)
        half = lax.broadcasted_iota(jnp.int32, (128, 2), 1)
        sel = jnp.logical_and(lane >= half * FD,
                              lane < half * FD + LD).astype(jnp.float32)
        zc = jnp.zeros((J, FD - LD), jnp.float32)
        zr0 = jnp.zeros((LD, 128), jnp.float32)
        zr1 = jnp.zeros((FD - LD - CD, 128), jnp.float32)
        zc_u = jnp.zeros((CD, FD - LD), jnp.float32)
        for b in range(BB):
            gb = g_ref[pl.ds(b * HB, HB), :]
            ut = u_ref[b]
            utw0 = jnp.concatenate([ut, zc_u, jnp.zeros((CD, FD), jnp.float32)],
                                   axis=1)
            utw1 = jnp.concatenate([jnp.zeros((CD, FD), jnp.float32), ut, zc_u],
                                   axis=1)
            w = jnp.concatenate([zr0, utw0, zr1, zr0, utw1, zr1], axis=0)
            v0 = jnp.concatenate([mp_ref[b], zc,
                                  jnp.zeros((J, FD), jnp.float32)], axis=1)
            v1 = jnp.concatenate([jnp.zeros((J, FD), jnp.float32),
                                  mp_ref[b], zc], axis=1)
            rid = lax.broadcasted_iota(jnp.int32, (J, 128), 0)
            mp2 = jnp.where(rid < J // 2, v0, v1)
            mpr_full = jnp.dot(expand, mp2,
                               preferred_element_type=jnp.float32)
            m_full = jnp.dot(gb, w, preferred_element_type=jnp.float32)
            t_full = gb * mpr_full
            prod = t_full * m_full
            nums = jnp.dot(prod, sel, preferred_element_type=jnp.float32)
            dens = jnp.dot(t_full, sel, preferred_element_type=jnp.float32)
            o = nums / dens
            out_ref[b] = jnp.concatenate([o[:, 0:1], o[:, 1:2]], axis=0)

    return pl.pallas_call(
        body,
        out_shape=jax.ShapeDtypeStruct((B, JK, 1), jnp.float32),
        grid=(B // BB,),
        in_specs=[
            pl.BlockSpec((BB * HB, 128), lambda i: (i, 0)),
            pl.BlockSpec((BB, J, LD), lambda i: (i, 0, 0)),
            pl.BlockSpec((BB, CD, LD), lambda i: (i, 0, 0)),
        ],
        out_specs=pl.BlockSpec((BB, JK, 1), lambda i: (i, 0, 0)),
    )(gpk, mp, u_selT)


def kernel(user_idx, motivation_proportion, user_purchase, beta,
           character_vector, u):
    pad = jnp.zeros((NUM_ITEMS, FD - LD - CD), jnp.float32)
    table = jnp.concatenate([beta.T, character_vector, pad], axis=1)
    flat_idx = (user_purchase.astype(jnp.int32)
                .reshape(B, 2, HB).transpose(0, 2, 1).reshape(N_TOT))
    u_flat = u.reshape(NUM_USER, LD * CD)
    gpk = _sc_gather(table, flat_idx).reshape(N_PK, 128)
    u_sel = _sc_gather_u(u_flat, user_idx.astype(jnp.int32))
    u_selT = u_sel.reshape(B, LD, CD).transpose(0, 2, 1)
    out = _tc_compute(gpk, motivation_proportion, u_selT)
    return out.reshape(B, J, K)


# speedup vs baseline: 1.5003x; 1.5003x over previous
import functools

import jax
import jax.numpy as jnp
from jax import lax
from jax.experimental import pallas as pl
from jax.experimental.pallas import tpu as pltpu
from jax.experimental.pallas import tpu_sc as plsc

NUM_USER = 100000
NUM_ITEMS = 100000
LD = 16
CD = 32
FD = 64
B, J, K = 1024, 20, 50
JK = J * K
N_TOT = B * JK
HB = JK // 2
N_PK = N_TOT // 2

NC, NS = 2, 16
NW = NC * NS
PER_W = N_TOT // NW
SUB = (128, 128, 64)
GROUP = sum(SUB)
N_GROUPS = PER_W // GROUP
U_PER_W = B // NW

BB = 16


def _sc_gather(table, flat_idx):
    mesh = plsc.VectorSubcoreMesh(core_axis_name="c", subcore_axis_name="s")

    @functools.partial(
        pl.kernel,
        out_type=jax.ShapeDtypeStruct((N_TOT, FD), jnp.float32),
        mesh=mesh,
        scratch_types=[
            pltpu.VMEM((PER_W,), jnp.int32),
            pltpu.VMEM((2, GROUP, FD), jnp.float32),
            pltpu.SemaphoreType.DMA,
            pltpu.SemaphoreType.DMA,
            pltpu.SemaphoreType.DMA,
            pltpu.SemaphoreType.DMA,
        ],
        compiler_params=pltpu.CompilerParams(use_tc_tiling_on_sc=False),
    )
    def gather_kernel(table_hbm, idx_hbm, out_hbm,
                      idx_v, rows_v, gsem0, gsem1, wsem0, wsem1):
        wid = lax.axis_index("s") * NC + lax.axis_index("c")
        gsems = (gsem0, gsem1)
        wsems = (wsem0, wsem1)

        wbase = pl.multiple_of(wid * PER_W, 8)
        pltpu.sync_copy(idx_hbm.at[pl.ds(wbase, PER_W)], idx_v)

        def issue_gather(g, s):
            off = 0
            for n in SUB:
                goff = pl.multiple_of(g * GROUP + off, 8)
                sl = idx_v.at[pl.ds(goff, n)]
                pltpu.async_copy(table_hbm.at[sl],
                                 rows_v.at[s, pl.ds(off, n)],
                                 gsems[s])
                off += n

        def drain_gather(s):
            pltpu.make_async_copy(table_hbm.at[pl.ds(0, GROUP)], rows_v.at[s],
                                  gsems[s]).wait()

        def issue_writeback(g, s):
            base = pl.multiple_of(wbase + g * GROUP, 8)
            pltpu.async_copy(rows_v.at[s], out_hbm.at[pl.ds(base, GROUP)],
                             wsems[s])

        def drain_writeback(s):
            pltpu.make_async_copy(table_hbm.at[pl.ds(0, GROUP)], rows_v.at[s],
                                  wsems[s]).wait()

        issue_gather(0, 0)

        def body(t, carry):
            for s in range(2):
                g = t * 2 + s
                drain_gather(s)
                issue_writeback(g, s)

                @pl.when(jnp.logical_and(g >= 1, g < N_GROUPS - 1))
                def _():
                    drain_writeback(1 - s)

                @pl.when(g < N_GROUPS - 1)
                def _():
                    issue_gather(g + 1, 1 - s)

            return carry

        lax.fori_loop(0, N_GROUPS // 2, body, 0)
        drain_writeback(0)
        drain_writeback(1)

    return gather_kernel(table, flat_idx)


def _sc_gather_u(u_flat, user_idx):
    mesh = plsc.VectorSubcoreMesh(core_axis_name="c", subcore_axis_name="s")

    @functools.partial(
        pl.kernel,
        out_type=jax.ShapeDtypeStruct((B, LD * CD), jnp.float32),
        mesh=mesh,
        scratch_types=[
            pltpu.VMEM((U_PER_W,), jnp.int32),
            pltpu.VMEM((U_PER_W, LD * CD), jnp.float32),
            pltpu.SemaphoreType.DMA,
        ],
    )
    def u_kernel(u_hbm, uidx_hbm, usel_hbm, uidx_v, urows_v, sem):
        wid = lax.axis_index("s") * NC + lax.axis_index("c")
        ubase = pl.multiple_of(wid * U_PER_W, 8)
        pltpu.sync_copy(uidx_hbm.at[pl.ds(ubase, U_PER_W)], uidx_v)
        pltpu.async_copy(u_hbm.at[uidx_v], urows_v, sem).wait()
        pltpu.sync_copy(urows_v, usel_hbm.at[pl.ds(ubase, U_PER_W)])

    return u_kernel(u_flat, user_idx)


def _tc_compute(gpk, mp, u_selT):
    def body(g_ref, mp_ref, u_ref, out_ref):
        rows_j = lax.broadcasted_iota(jnp.int32, (HB, J), 0) // (K // 2)
        cols_j = lax.broadcasted_iota(jnp.int32, (HB, J), 1)
        expand = (rows_j == cols_j).astype(jnp.float32)
        lane = lax.broadcasted_iota(jnp.int32, (128, 2), 0)
        half = lax.broadcasted_iota(jnp.int32, (128, 2), 1)
        sel = jnp.logical_and(lane >= half * FD,
                              lane < half * FD + LD).astype(jnp.float32)
        zc = jnp.zeros((J, FD - LD), jnp.float32)
        zr0 = jnp.zeros((LD, 128), jnp.float32)
        zr1 = jnp.zeros((FD - LD - CD, 128), jnp.float32)
        zc_u = jnp.zeros((CD, FD - LD), jnp.float32)
        for b in range(BB):
            gb = g_ref[pl.ds(b * HB, HB), :]
            ut = u_ref[b]
            utw0 = jnp.concatenate([ut, zc_u, jnp.zeros((CD, FD), jnp.float32)],
                                   axis=1)
            utw1 = jnp.concatenate([jnp.zeros((CD, FD), jnp.float32), ut, zc_u],
                                   axis=1)
            w = jnp.concatenate([zr0, utw0, zr1, zr0, utw1, zr1], axis=0)
            mp2 = jnp.concatenate([mp_ref[b], zc, mp_ref[b], zc], axis=1)
            mpr_full = jnp.dot(expand, mp2,
                               preferred_element_type=jnp.float32)
            m_full = jnp.dot(gb, w, preferred_element_type=jnp.float32)
            t_full = gb * mpr_full
            prod = t_full * m_full
            nums = jnp.dot(prod, sel, preferred_element_type=jnp.float32)
            dens = jnp.dot(t_full, sel, preferred_element_type=jnp.float32)
            out_ref[b] = nums / dens

    return pl.pallas_call(
        body,
        out_shape=jax.ShapeDtypeStruct((B, HB, 2), jnp.float32),
        grid=(B // BB,),
        in_specs=[
            pl.BlockSpec((BB * HB, 128), lambda i: (i, 0)),
            pl.BlockSpec((BB, J, LD), lambda i: (i, 0, 0)),
            pl.BlockSpec((BB, CD, LD), lambda i: (i, 0, 0)),
        ],
        out_specs=pl.BlockSpec((BB, HB, 2), lambda i: (i, 0, 0)),
    )(gpk, mp, u_selT)


def kernel(user_idx, motivation_proportion, user_purchase, beta,
           character_vector, u):
    pad = jnp.zeros((NUM_ITEMS, FD - LD - CD), jnp.float32)
    table = jnp.concatenate([beta.T, character_vector, pad], axis=1)
    flat_idx = user_purchase.reshape(N_TOT).astype(jnp.int32)
    u_flat = u.reshape(NUM_USER, LD * CD)
    gpk = _sc_gather(table, flat_idx).reshape(N_PK, 128)
    u_sel = _sc_gather_u(u_flat, user_idx.astype(jnp.int32))
    u_selT = u_sel.reshape(B, LD, CD).transpose(0, 2, 1)
    out = _tc_compute(gpk, motivation_proportion, u_selT)
    return out.reshape(B, J, K)


# speedup vs baseline: 1.5228x; 1.0150x over previous
import functools

import jax
import jax.numpy as jnp
from jax import lax
from jax.experimental import pallas as pl
from jax.experimental.pallas import tpu as pltpu
from jax.experimental.pallas import tpu_sc as plsc

NUM_USER = 100000
NUM_ITEMS = 100000
LD = 16
CD = 32
FD = 64
B, J, K = 1024, 20, 50
JK = J * K
N_TOT = B * JK
HB = JK // 2
N_PK = N_TOT // 2

NC, NS = 2, 16
NW = NC * NS
PER_W = N_TOT // NW
SUB = (128, 128, 128, 128, 128)
GROUP = sum(SUB)
N_GROUPS = PER_W // GROUP
U_PER_W = B // NW

BB = 16


def _sc_gather(table, flat_idx):
    mesh = plsc.VectorSubcoreMesh(core_axis_name="c", subcore_axis_name="s")

    @functools.partial(
        pl.kernel,
        out_type=jax.ShapeDtypeStruct((N_TOT, FD), jnp.float32),
        mesh=mesh,
        scratch_types=[
            pltpu.VMEM((PER_W,), jnp.int32),
            pltpu.VMEM((2, GROUP, FD), jnp.float32),
            pltpu.SemaphoreType.DMA,
            pltpu.SemaphoreType.DMA,
            pltpu.SemaphoreType.DMA,
            pltpu.SemaphoreType.DMA,
        ],
        compiler_params=pltpu.CompilerParams(use_tc_tiling_on_sc=False),
    )
    def gather_kernel(table_hbm, idx_hbm, out_hbm,
                      idx_v, rows_v, gsem0, gsem1, wsem0, wsem1):
        wid = lax.axis_index("s") * NC + lax.axis_index("c")
        gsems = (gsem0, gsem1)
        wsems = (wsem0, wsem1)

        wbase = pl.multiple_of(wid * PER_W, 8)
        pltpu.sync_copy(idx_hbm.at[pl.ds(wbase, PER_W)], idx_v)

        def issue_gather(g, s):
            off = 0
            for n in SUB:
                goff = pl.multiple_of(g * GROUP + off, 8)
                sl = idx_v.at[pl.ds(goff, n)]
                pltpu.async_copy(table_hbm.at[sl],
                                 rows_v.at[s, pl.ds(off, n)],
                                 gsems[s])
                off += n

        def drain_gather(s):
            pltpu.make_async_copy(table_hbm.at[pl.ds(0, GROUP)], rows_v.at[s],
                                  gsems[s]).wait()

        def issue_writeback(g, s):
            base = pl.multiple_of(wbase + g * GROUP, 8)
            pltpu.async_copy(rows_v.at[s], out_hbm.at[pl.ds(base, GROUP)],
                             wsems[s])

        def drain_writeback(s):
            pltpu.make_async_copy(table_hbm.at[pl.ds(0, GROUP)], rows_v.at[s],
                                  wsems[s]).wait()

        issue_gather(0, 0)

        def body(t, carry):
            for s in range(2):
                g = t * 2 + s
                drain_gather(s)
                issue_writeback(g, s)

                @pl.when(jnp.logical_and(g >= 1, g < N_GROUPS - 1))
                def _():
                    drain_writeback(1 - s)

                @pl.when(g < N_GROUPS - 1)
                def _():
                    issue_gather(g + 1, 1 - s)

            return carry

        lax.fori_loop(0, N_GROUPS // 2, body, 0)
        drain_writeback(0)
        drain_writeback(1)

    return gather_kernel(table, flat_idx)


def _sc_gather_u(u_flat, user_idx):
    mesh = plsc.VectorSubcoreMesh(core_axis_name="c", subcore_axis_name="s")

    @functools.partial(
        pl.kernel,
        out_type=jax.ShapeDtypeStruct((B, LD * CD), jnp.float32),
        mesh=mesh,
        scratch_types=[
            pltpu.VMEM((U_PER_W,), jnp.int32),
            pltpu.VMEM((U_PER_W, LD * CD), jnp.float32),
            pltpu.SemaphoreType.DMA,
        ],
    )
    def u_kernel(u_hbm, uidx_hbm, usel_hbm, uidx_v, urows_v, sem):
        wid = lax.axis_index("s") * NC + lax.axis_index("c")
        ubase = pl.multiple_of(wid * U_PER_W, 8)
        pltpu.sync_copy(uidx_hbm.at[pl.ds(ubase, U_PER_W)], uidx_v)
        pltpu.async_copy(u_hbm.at[uidx_v], urows_v, sem).wait()
        pltpu.sync_copy(urows_v, usel_hbm.at[pl.ds(ubase, U_PER_W)])

    return u_kernel(u_flat, user_idx)


def _tc_compute(gpk, mp, u_selT):
    def body(g_ref, mp_ref, u_ref, out_ref):
        rows_j = lax.broadcasted_iota(jnp.int32, (HB, J), 0) // (K // 2)
        cols_j = lax.broadcasted_iota(jnp.int32, (HB, J), 1)
        expand = (rows_j == cols_j).astype(jnp.float32)
        lane = lax.broadcasted_iota(jnp.int32, (128, 2), 0)
        half = lax.broadcasted_iota(jnp.int32, (128, 2), 1)
        sel = jnp.logical_and(lane >= half * FD,
                              lane < half * FD + LD).astype(jnp.float32)
        zc = jnp.zeros((J, FD - LD), jnp.float32)
        zr0 = jnp.zeros((LD, 128), jnp.float32)
        zr1 = jnp.zeros((FD - LD - CD, 128), jnp.float32)
        zc_u = jnp.zeros((CD, FD - LD), jnp.float32)
        for b in range(BB):
            gb = g_ref[pl.ds(b * HB, HB), :]
            ut = u_ref[b]
            utw0 = jnp.concatenate([ut, zc_u, jnp.zeros((CD, FD), jnp.float32)],
                                   axis=1)
            utw1 = jnp.concatenate([jnp.zeros((CD, FD), jnp.float32), ut, zc_u],
                                   axis=1)
            w = jnp.concatenate([zr0, utw0, zr1, zr0, utw1, zr1], axis=0)
            mp2 = jnp.concatenate([mp_ref[b], zc, mp_ref[b], zc], axis=1)
            mpr_full = jnp.dot(expand, mp2,
                               preferred_element_type=jnp.float32)
            m_full = jnp.dot(gb, w, preferred_element_type=jnp.float32)
            t_full = gb * mpr_full
            prod = t_full * m_full
            nums = jnp.dot(prod, sel, preferred_element_type=jnp.float32)
            dens = jnp.dot(t_full, sel, preferred_element_type=jnp.float32)
            out_ref[b] = nums / dens

    return pl.pallas_call(
        body,
        out_shape=jax.ShapeDtypeStruct((B, HB, 2), jnp.float32),
        grid=(B // BB,),
        in_specs=[
            pl.BlockSpec((BB * HB, 128), lambda i: (i, 0)),
            pl.BlockSpec((BB, J, LD), lambda i: (i, 0, 0)),
            pl.BlockSpec((BB, CD, LD), lambda i: (i, 0, 0)),
        ],
        out_specs=pl.BlockSpec((BB, HB, 2), lambda i: (i, 0, 0)),
    )(gpk, mp, u_selT)


def kernel(user_idx, motivation_proportion, user_purchase, beta,
           character_vector, u):
    pad = jnp.zeros((NUM_ITEMS, FD - LD - CD), jnp.float32)
    table = jnp.concatenate([beta.T, character_vector, pad], axis=1)
    flat_idx = user_purchase.reshape(N_TOT).astype(jnp.int32)
    u_flat = u.reshape(NUM_USER, LD * CD)
    gpk = _sc_gather(table, flat_idx).reshape(N_PK, 128)
    u_sel = _sc_gather_u(u_flat, user_idx.astype(jnp.int32))
    u_selT = u_sel.reshape(B, LD, CD).transpose(0, 2, 1)
    out = _tc_compute(gpk, motivation_proportion, u_selT)
    return out.reshape(B, J, K)
